# baseline (speedup 1.0000x reference)
import jax
import jax.numpy as jnp
from jax.experimental import pallas as pl
from jax.experimental.pallas import tpu as pltpu

_N_LAYERS = 3
_PER_LAYER_OPS = 10


def _dot(a, b):
    return jax.lax.dot_general(a, b, (((1,), (0,)), ((), ())),
                               preferred_element_type=jnp.float32)


def _dot_t(a, b):
    return jax.lax.dot_general(a, b, (((1,), (1,)), ((), ())),
                               preferred_element_type=jnp.float32)


def _row(v):
    return v[None, :]


def _fused_body(*refs):
    out_ref = refs[-3]
    x_scratch = refs[-2]
    sem = refs[-1]
    in_refs = refs[:-3]
    it = iter(in_refs)
    x_hbm = next(it)
    n_rows_total = x_hbm.shape[0]
    chunk = n_rows_total // 4
    copies = []
    for i in range(4):
        lo = i * chunk
        hi = n_rows_total if i == 3 else (i + 1) * chunk
        cp = pltpu.make_async_copy(x_hbm.at[pl.ds(lo, hi - lo), :],
                                   x_scratch.at[pl.ds(lo, hi - lo), :], sem)
        copies.append(cp)
    for cp in copies:
        cp.start()
    for cp in copies:
        cp.wait()
    x = x_scratch[...]
    proj_W = next(it)[...]
    proj_b = _row(next(it)[...])
    mem = next(it)[...]
    layers = []
    for _ in range(_N_LAYERS):
        Wih = next(it)[...]
        bih = _row(next(it)[...])
        Whh = next(it)[...]
        bhh = _row(next(it)[...])
        msg_W1 = next(it)[...]
        msg_b1 = _row(next(it)[...])
        msg_W2 = next(it)[...]
        msg_b2 = _row(next(it)[...])
        agg_W = next(it)[...]
        agg_b = _row(next(it)[...])
        layers.append((Wih, bih, Whh, bhh, msg_W1, msg_b1,
                       msg_W2, msg_b2, agg_W, agg_b))
    cls_W1 = next(it)[...]
    cls_b1 = _row(next(it)[...])
    cls_W2 = next(it)[...]
    cls_b2 = next(it)[0]

    d_h = proj_W.shape[0]
    d_mem = mem.shape[1]
    n_rows = x.shape[0]
    ones_row = jnp.ones((1, n_rows), jnp.float32)
    inv_n = 1.0 / n_rows

    a = x
    Mt = proj_W
    c = proj_b
    hbar = _dot_t(_dot(ones_row, x), Mt) * inv_n + c
    for l in range(_N_LAYERS):
        (Wih, bih, Whh, bhh, msg_W1, msg_b1,
         msg_W2, msg_b2, agg_W, agg_b) = layers[l]

        W1h = msg_W1[:, :d_h]
        G = _dot(W1h, Mt)
        P = _dot_t(a, G)

        gi = _dot_t(hbar, Wih) + bih
        gh = _dot_t(mem, Whh) + bhh
        r = jax.nn.sigmoid(gi[:, 0:d_mem] + gh[:, 0:d_mem])
        z = jax.nn.sigmoid(gi[:, d_mem:2 * d_mem] + gh[:, d_mem:2 * d_mem])
        nn = jnp.tanh(gi[:, 2 * d_mem:] + r * gh[:, 2 * d_mem:])
        mem = (1.0 - z) * nn + z * mem

        mvec = _dot_t(mem, msg_W1[:, d_h:]) + msg_b1
        g = _dot_t(c, W1h) + mvec
        a = jax.nn.relu(P + g)
        Mt = _dot(agg_W, msg_W2)
        c = _dot_t(msg_b2, agg_W) + agg_b
        if l + 1 < _N_LAYERS:
            hbar = _dot_t(_dot(ones_row, a), Mt) * inv_n + c

    Gc = _dot(cls_W1, Mt)
    gc = _dot_t(c, cls_W1) + cls_b1
    c1 = jax.nn.relu(_dot_t(a, Gc) + gc)
    W2pad = jnp.concatenate(
        [cls_W2, jnp.zeros((d_h - 1, cls_W2.shape[1]), jnp.float32)], axis=0)
    out128 = _dot_t(c1, W2pad)
    out_ref[...] = out128[:, 0:1] + cls_b2


def kernel(x, edge_index, edge_attr, edge_time, params):
    p = params
    operands = [x, p['proj_W'], p['proj_b'], p['memory']]
    for lp in p['layers']:
        operands += [lp['Wih'], lp['bih'], lp['Whh'], lp['bhh'],
                     lp['msg_W1'], lp['msg_b1'], lp['msg_W2'], lp['msg_b2'],
                     lp['agg_W'], lp['agg_b']]
    operands += [p['cls_W1'], p['cls_b1'], p['cls_W2'], p['cls_b2']]

    vmem = pl.BlockSpec(memory_space=pltpu.MemorySpace.VMEM)
    smem = pl.BlockSpec(memory_space=pltpu.MemorySpace.SMEM)
    any_spec = pl.BlockSpec(memory_space=pl.ANY)
    in_specs = [any_spec] + [vmem] * (len(operands) - 2) + [smem]

    return pl.pallas_call(
        _fused_body,
        in_specs=in_specs,
        out_shape=jax.ShapeDtypeStruct((x.shape[0], 1), jnp.float32),
        scratch_shapes=[pltpu.VMEM(x.shape, x.dtype),
                        pltpu.SemaphoreType.DMA],
    )(*operands)


# speedup vs baseline: 1.0341x; 1.0341x over previous
import jax
import jax.numpy as jnp
from jax.experimental import pallas as pl
from jax.experimental.pallas import tpu as pltpu

_N_LAYERS = 3
_PER_LAYER_OPS = 10


def _dot(a, b):
    return jax.lax.dot_general(a, b, (((1,), (0,)), ((), ())),
                               preferred_element_type=jnp.float32)


def _dot_t(a, b):
    return jax.lax.dot_general(a, b, (((1,), (1,)), ((), ())),
                               preferred_element_type=jnp.float32)


def _row(v):
    return v[None, :]


def _fused_body(*refs):
    out_ref = refs[-1]
    in_refs = refs[:-1]
    it = iter(in_refs)
    x = next(it)[...]
    proj_W = next(it)[...]
    proj_b = _row(next(it)[...])
    mem = next(it)[...]
    layers = []
    for _ in range(_N_LAYERS):
        Wih = next(it)[...]
        bih = _row(next(it)[...])
        Whh = next(it)[...]
        bhh = _row(next(it)[...])
        msg_W1 = next(it)[...]
        msg_b1 = _row(next(it)[...])
        msg_W2 = next(it)[...]
        msg_b2 = _row(next(it)[...])
        agg_W = next(it)[...]
        agg_b = _row(next(it)[...])
        layers.append((Wih, bih, Whh, bhh, msg_W1, msg_b1,
                       msg_W2, msg_b2, agg_W, agg_b))
    cls_W1 = next(it)[...]
    cls_b1 = _row(next(it)[...])
    cls_W2 = next(it)[...]
    cls_b2 = next(it)[0]

    d_h = proj_W.shape[0]
    d_mem = mem.shape[1]
    n_rows = x.shape[0]
    ones_row = jnp.ones((1, n_rows), jnp.float32)
    inv_n = 1.0 / n_rows

    a = x
    Mt = proj_W
    c = proj_b
    hbar = _dot_t(_dot(ones_row, x), Mt) * inv_n + c
    for l in range(_N_LAYERS):
        (Wih, bih, Whh, bhh, msg_W1, msg_b1,
         msg_W2, msg_b2, agg_W, agg_b) = layers[l]

        W1h = msg_W1[:, :d_h]
        G = _dot(W1h, Mt)
        P = _dot_t(a, G)

        gi = _dot_t(hbar, Wih) + bih
        gh = _dot_t(mem, Whh) + bhh
        r = jax.nn.sigmoid(gi[:, 0:d_mem] + gh[:, 0:d_mem])
        z = jax.nn.sigmoid(gi[:, d_mem:2 * d_mem] + gh[:, d_mem:2 * d_mem])
        nn = jnp.tanh(gi[:, 2 * d_mem:] + r * gh[:, 2 * d_mem:])
        mem = (1.0 - z) * nn + z * mem

        mvec = _dot_t(mem, msg_W1[:, d_h:]) + msg_b1
        g = _dot_t(c, W1h) + mvec
        a = jax.nn.relu(P + g)
        Mt = _dot(agg_W, msg_W2)
        c = _dot_t(msg_b2, agg_W) + agg_b
        if l + 1 < _N_LAYERS:
            hbar = _dot_t(_dot(ones_row, a), Mt) * inv_n + c

    Gc = _dot(cls_W1, Mt)
    gc = _dot_t(c, cls_W1) + cls_b1
    c1 = jax.nn.relu(_dot_t(a, Gc) + gc)
    W2pad = jnp.concatenate(
        [cls_W2, jnp.zeros((d_h - 1, cls_W2.shape[1]), jnp.float32)], axis=0)
    out128 = _dot_t(c1, W2pad)
    out_ref[...] = out128[:, 0:1] + cls_b2


def kernel(x, edge_index, edge_attr, edge_time, params):
    p = params
    operands = [x, p['proj_W'], p['proj_b'], p['memory']]
    for lp in p['layers']:
        operands += [lp['Wih'], lp['bih'], lp['Whh'], lp['bhh'],
                     lp['msg_W1'], lp['msg_b1'], lp['msg_W2'], lp['msg_b2'],
                     lp['agg_W'], lp['agg_b']]
    operands += [p['cls_W1'], p['cls_b1'], p['cls_W2'], p['cls_b2']]

    vmem = pl.BlockSpec(memory_space=pltpu.MemorySpace.VMEM)
    smem = pl.BlockSpec(memory_space=pltpu.MemorySpace.SMEM)
    in_specs = [vmem] * (len(operands) - 1) + [smem]

    return pl.pallas_call(
        _fused_body,
        in_specs=in_specs,
        out_shape=jax.ShapeDtypeStruct((x.shape[0], 1), jnp.float32),
    )(*operands)
